# baseline (speedup 1.0000x reference)
import functools

import jax
import jax.numpy as jnp
from jax import lax
from jax.experimental import pallas as pl
from jax.experimental.pallas import tpu as pltpu
from jax.experimental.pallas import tpu_sc as plsc

VOCAB_ROWS = 500
DIM = 64

_info = plsc.get_sparse_core_info()
NC, NS = _info.num_cores, _info.num_subcores
NW = NC * NS

CHUNK = 200


def _make_gather(n_batch: int, seq: int):
    assert seq == CHUNK
    total_rows = n_batch * seq
    assert total_rows % (NW * 2 * CHUNK) == 0
    rows_per_w = total_rows // NW
    slabs_per_w = rows_per_w // seq
    n_pairs = rows_per_w // (2 * CHUNK)
    mesh = plsc.VectorSubcoreMesh(core_axis_name="c", subcore_axis_name="s")

    @functools.partial(
        pl.kernel,
        mesh=mesh,
        out_type=jax.ShapeDtypeStruct((n_batch, seq, DIM), jnp.float32),
        scratch_types=[
            pltpu.VMEM_SHARED((2 * VOCAB_ROWS, DIM), jnp.float32),
            pltpu.VMEM((4 * CHUNK,), jnp.int32),
            pltpu.VMEM((2, 2 * CHUNK, DIM), jnp.float32),
            pltpu.SemaphoreType.DMA,
            pltpu.SemaphoreType.DMA,
            pltpu.SemaphoreType.DMA,
            pltpu.SemaphoreType.DMA,
        ],
    )
    def gather_kernel(idx2_hbm, table_hbm, out_hbm, table_sp, idx_v, rows_v,
                      sem_g0, sem_g1, sem_o0, sem_o1):
        cid = lax.axis_index("c")
        sid = lax.axis_index("s")
        wid = sid * NC + cid
        w_base = wid * rows_per_w
        w_slab = wid * slabs_per_w

        @pl.when(sid == 0)
        def _():
            pltpu.sync_copy(table_hbm, table_sp.at[pl.ds(0, VOCAB_ROWS)])

        plsc.subcore_barrier()

        def load_idx(pair):
            pltpu.sync_copy(
                idx2_hbm.at[pl.ds(2 * (w_base + pair * 2 * CHUNK), 4 * CHUNK)],
                idx_v)

        def start_gather(half, sem):
            return pltpu.async_copy(
                table_sp.at[idx_v.at[pl.ds(half * 2 * CHUNK, 2 * CHUNK)]],
                rows_v.at[half], sem)

        def start_store(pair, half, sem):
            return pltpu.async_copy(
                rows_v.at[half, pl.ds(0, CHUNK)],
                out_hbm.at[w_slab + pair * 2 + half],
                sem)

        def wait_store(half, sem):
            pltpu.make_async_copy(
                rows_v.at[half, pl.ds(0, CHUNK)],
                out_hbm.at[w_slab], sem).wait()

        load_idx(0)
        start_gather(0, sem_g0).wait()
        h1 = start_gather(1, sem_g1)
        start_store(0, 0, sem_o0)
        h1.wait()
        start_store(0, 1, sem_o1)

        def body(g, _):
            load_idx(g)
            wait_store(0, sem_o0)
            h0 = start_gather(0, sem_g0)
            h0.wait()
            wait_store(1, sem_o1)
            h1 = start_gather(1, sem_g1)
            start_store(g, 0, sem_o0)
            h1.wait()
            start_store(g, 1, sem_o1)
            return ()

        lax.fori_loop(1, n_pairs, body, (), unroll=False)

        wait_store(0, sem_o0)
        wait_store(1, sem_o1)

    return gather_kernel


def kernel(x, table):
    b, s = x.shape
    idx2 = jnp.repeat(x.reshape(b * s).astype(jnp.int32) * 2, 2, axis=0)
    eff_table = table.at[0].set(0.0)
    return _make_gather(b, s)(idx2, eff_table)


# speedup vs baseline: 1.9793x; 1.9793x over previous
import functools

import jax
import jax.numpy as jnp
from jax import lax
from jax.experimental import pallas as pl
from jax.experimental.pallas import tpu as pltpu
from jax.experimental.pallas import tpu_sc as plsc

VOCAB_ROWS = 500
DIM = 64

_info = plsc.get_sparse_core_info()
NC, NS = _info.num_cores, _info.num_subcores
NW = NC * NS

CHUNK = 512


def _make_gather(total_rows: int):
    assert total_rows % (NW * 2 * CHUNK) == 0
    rows_per_w = total_rows // NW
    n_pairs = rows_per_w // (2 * CHUNK)
    mesh = plsc.VectorSubcoreMesh(core_axis_name="c", subcore_axis_name="s")

    @functools.partial(
        pl.kernel,
        mesh=mesh,
        compiler_params=pltpu.CompilerParams(use_tc_tiling_on_sc=False),
        out_type=jax.ShapeDtypeStruct((total_rows, DIM), jnp.float32),
        scratch_types=[
            pltpu.VMEM_SHARED((VOCAB_ROWS, DIM), jnp.float32),
            pltpu.VMEM((2 * CHUNK,), jnp.int32),
            pltpu.VMEM((2, CHUNK, DIM), jnp.float32),
            pltpu.SemaphoreType.DMA,
            pltpu.SemaphoreType.DMA,
            pltpu.SemaphoreType.DMA,
            pltpu.SemaphoreType.DMA,
        ],
    )
    def gather_kernel(idx_hbm, table_hbm, out_hbm, table_sp, idx_v, rows_v,
                      sem_g0, sem_g1, sem_o0, sem_o1):
        cid = lax.axis_index("c")
        sid = lax.axis_index("s")
        wid = sid * NC + cid
        w_base = wid * rows_per_w

        @pl.when(sid == 0)
        def _():
            pltpu.sync_copy(table_hbm, table_sp)

        plsc.subcore_barrier()

        def load_idx(pair):
            pltpu.sync_copy(
                idx_hbm.at[pl.ds(w_base + pair * 2 * CHUNK, 2 * CHUNK)], idx_v)

        def start_gather(half, sem):
            return pltpu.async_copy(
                table_sp.at[idx_v.at[pl.ds(half * CHUNK, CHUNK)]],
                rows_v.at[half], sem)

        def start_store(pair, half, sem):
            return pltpu.async_copy(
                rows_v.at[half],
                out_hbm.at[pl.ds(w_base + (pair * 2 + half) * CHUNK, CHUNK)],
                sem)

        def wait_store(half, sem):
            pltpu.make_async_copy(
                rows_v.at[half], out_hbm.at[pl.ds(w_base, CHUNK)], sem).wait()

        load_idx(0)
        start_gather(0, sem_g0).wait()
        h1 = start_gather(1, sem_g1)
        start_store(0, 0, sem_o0)
        h1.wait()
        start_store(0, 1, sem_o1)

        def body(g, _):
            load_idx(g)
            wait_store(0, sem_o0)
            h0 = start_gather(0, sem_g0)
            h0.wait()
            wait_store(1, sem_o1)
            h1 = start_gather(1, sem_g1)
            start_store(g, 0, sem_o0)
            h1.wait()
            start_store(g, 1, sem_o1)
            return ()

        lax.fori_loop(1, n_pairs, body, (), unroll=False)

        wait_store(0, sem_o0)
        wait_store(1, sem_o1)

    return gather_kernel


def kernel(x, table):
    b, s = x.shape
    idx = x.reshape(b * s).astype(jnp.int32)
    eff_table = table.at[0].set(0.0)
    out = _make_gather(b * s)(idx, eff_table)
    return out.reshape(b, s, DIM)
